# baseline (speedup 1.0000x reference)
import functools

import jax
import jax.numpy as jnp
from jax import lax
from jax.experimental import pallas as pl
from jax.experimental.pallas import tpu as pltpu
from jax.experimental.pallas import tpu_sc as plsc

B = 1024
F = 64
NC = 2
NS = 16
NW = NC * NS
BPW = B // NW

_mesh = plsc.VectorSubcoreMesh(core_axis_name="c", subcore_axis_name="s")

_GATHER_DN = lax.GatherDimensionNumbers(
    offset_dims=(), collapsed_slice_dims=(0,), start_index_map=(0,))


def _permute(x, idx):
    return lax.gather(x, idx[:, None], _GATHER_DN, (1,),
                      mode=lax.GatherScatterMode.PROMISE_IN_BOUNDS)


@functools.partial(
    pl.kernel,
    mesh=_mesh,
    out_type=[
        jax.ShapeDtypeStruct((B,), jnp.float32),
        jax.ShapeDtypeStruct((B,), jnp.float32),
    ],
    scratch_types=[
        pltpu.VMEM((BPW,), jnp.int32),
        pltpu.VMEM((BPW,), jnp.int32),
        pltpu.VMEM((BPW, F), jnp.float32),
        pltpu.VMEM((BPW, F), jnp.float32),
        pltpu.VMEM((BPW,), jnp.float32),
        pltpu.VMEM((BPW,), jnp.float32),
        pltpu.VMEM((BPW,), jnp.float32),
        pltpu.VMEM((BPW,), jnp.float32),
        pltpu.SemaphoreType.DMA,
    ],
    compiler_params=pltpu.CompilerParams(use_tc_tiling_on_sc=False),
)
def _sc_gather_dot(users_hbm, items_hbm, uemb_hbm, iemb_hbm, ub_hbm, ib_hbm,
                   r_hbm, d_hbm,
                   uidx, iidx, ues, uis, ub, ib, rloc, dloc, sem):
    wid = lax.axis_index("s") * NC + lax.axis_index("c")
    base = wid * BPW
    pltpu.sync_copy(users_hbm.at[pl.ds(base, BPW)], uidx)
    pltpu.sync_copy(items_hbm.at[pl.ds(base, BPW)], iidx)
    cp1 = pltpu.async_copy(uemb_hbm.at[uidx], ues, sem)
    cp2 = pltpu.async_copy(iemb_hbm.at[iidx], uis, sem)
    cp3 = pltpu.async_copy(ub_hbm.at[uidx], ub, sem)
    cp4 = pltpu.async_copy(ib_hbm.at[iidx], ib, sem)
    cp1.wait()
    cp2.wait()
    cp3.wait()
    cp4.wait()
    iota = lax.iota(jnp.int32, 16)
    for g in range(BPW // 16):
        dvec = jnp.zeros((16,), jnp.float32)
        for i in range(16):
            r_i = g * 16 + i
            acc = ues[r_i, pl.ds(0, 16)] * uis[r_i, pl.ds(0, 16)]
            for k in range(1, F // 16):
                acc = acc + ues[r_i, pl.ds(16 * k, 16)] * uis[r_i, pl.ds(16 * k, 16)]
            for sh in (8, 4, 2, 1):
                acc = acc + _permute(acc, iota ^ sh)
            dvec = jnp.where(iota == i, acc, dvec)
        dloc[pl.ds(g * 16, 16)] = dvec
        rloc[pl.ds(g * 16, 16)] = ub[pl.ds(g * 16, 16)] + ib[pl.ds(g * 16, 16)]
    pltpu.sync_copy(rloc, r_hbm.at[pl.ds(base, BPW)])
    pltpu.sync_copy(dloc, d_hbm.at[pl.ds(base, BPW)])


def _tc_body(r_ref, d_ref, o_ref):
    o_ref[...] = r_ref[...] + d_ref[...]


def kernel(users, items, user_emb, item_emb, user_bias, item_bias):
    users = users.astype(jnp.int32)
    items = items.astype(jnp.int32)
    r, d = _sc_gather_dot(users, items, user_emb, item_emb,
                          user_bias.reshape(-1), item_bias.reshape(-1))
    out = pl.pallas_call(
        _tc_body,
        out_shape=jax.ShapeDtypeStruct((B, B), jnp.float32),
    )(r.reshape(B, 1), d.reshape(1, B))
    return out


# speedup vs baseline: 8.3104x; 8.3104x over previous
import functools

import jax
import jax.numpy as jnp
from jax import lax
from jax.experimental import pallas as pl
from jax.experimental.pallas import tpu as pltpu
from jax.experimental.pallas import tpu_sc as plsc

B = 1024
F = 64
WIN = 128
NBUF = 4
NC = 2
NS = 16
NW = NC * NS
BPW = B // NW

_mesh = plsc.VectorSubcoreMesh(core_axis_name="c", subcore_axis_name="s")

_GATHER_DN = lax.GatherDimensionNumbers(
    offset_dims=(), collapsed_slice_dims=(0,), start_index_map=(0,))


def _permute(x, idx):
    return lax.gather(x, idx[:, None], _GATHER_DN, (1,),
                      mode=lax.GatherScatterMode.PROMISE_IN_BOUNDS)


@functools.partial(
    pl.kernel,
    mesh=_mesh,
    out_type=[
        jax.ShapeDtypeStruct((B,), jnp.float32),
        jax.ShapeDtypeStruct((B,), jnp.float32),
    ],
    scratch_types=[
        pltpu.VMEM((BPW,), jnp.int32),
        pltpu.VMEM((BPW,), jnp.int32),
        pltpu.VMEM((NBUF, F, WIN), jnp.float32),
        pltpu.VMEM((NBUF, F, WIN), jnp.float32),
        pltpu.VMEM((BPW,), jnp.float32),
        pltpu.VMEM((BPW,), jnp.float32),
        pltpu.VMEM((BPW,), jnp.float32),
        pltpu.VMEM((BPW,), jnp.float32),
        pltpu.SemaphoreType.DMA,
        pltpu.SemaphoreType.DMA,
    ],
    compiler_params=pltpu.CompilerParams(needs_layout_passes=False),
)
def _sc_gather_dot(users_hbm, items_hbm, uembt_hbm, iembt_hbm, ub_hbm, ib_hbm,
                   r_hbm, d_hbm,
                   uidx, iidx, ublk, iblk, ub, ib, rloc, dloc, sem, bsem):
    wid = lax.axis_index("s") * NC + lax.axis_index("c")
    base = wid * BPW
    pltpu.sync_copy(users_hbm.at[pl.ds(base, BPW)], uidx)
    pltpu.sync_copy(items_hbm.at[pl.ds(base, BPW)], iidx)
    cpu_b = pltpu.async_copy(ub_hbm.at[uidx], ub, bsem)
    cpi_b = pltpu.async_copy(ib_hbm.at[iidx], ib, bsem)
    iota = lax.iota(jnp.int32, 16)
    uvecs = [uidx[pl.ds(0, 16)], uidx[pl.ds(16, 16)]]
    ivecs = [iidx[pl.ds(0, 16)], iidx[pl.ds(16, 16)]]

    def issue(j):
        ru = uvecs[j // 16][j % 16]
        ri = ivecs[j // 16][j % 16]
        off_u = pl.multiple_of(ru & -WIN, WIN)
        off_i = pl.multiple_of(ri & -WIN, WIN)
        s = j % NBUF
        return (pltpu.async_copy(uembt_hbm.at[:, pl.ds(off_u, WIN)],
                                 ublk.at[s], sem),
                pltpu.async_copy(iembt_hbm.at[:, pl.ds(off_i, WIN)],
                                 iblk.at[s], sem))

    pending = [issue(j) for j in range(NBUF - 1)]
    dvecs = [jnp.zeros((16,), jnp.float32), jnp.zeros((16,), jnp.float32)]
    for j in range(BPW):
        if j + NBUF - 1 < BPW:
            pending.append(issue(j + NBUF - 1))
        cu_cp, ci_cp = pending[j]
        cu_cp.wait()
        ci_cp.wait()
        s = j % NBUF
        cu = jnp.full((16,), 0, jnp.int32) + (uvecs[j // 16][j % 16] & (WIN - 1))
        ci = jnp.full((16,), 0, jnp.int32) + (ivecs[j // 16][j % 16] & (WIN - 1))
        acc = jnp.zeros((16,), jnp.float32)
        for k in range(F // 16):
            rows = iota + k * 16
            acc = acc + (plsc.load_gather(ublk.at[s], [rows, cu])
                         * plsc.load_gather(iblk.at[s], [rows, ci]))
        for sh in (8, 4, 2, 1):
            acc = acc + _permute(acc, iota ^ sh)
        dvecs[j // 16] = jnp.where(iota == (j % 16), acc, dvecs[j // 16])
    dloc[pl.ds(0, 16)] = dvecs[0]
    dloc[pl.ds(16, 16)] = dvecs[1]
    cpu_b.wait()
    cpi_b.wait()
    for g in range(BPW // 16):
        rloc[pl.ds(g * 16, 16)] = (ub[pl.ds(g * 16, 16)]
                                   + ib[pl.ds(g * 16, 16)])
    pltpu.sync_copy(rloc, r_hbm.at[pl.ds(base, BPW)])
    pltpu.sync_copy(dloc, d_hbm.at[pl.ds(base, BPW)])


def _tc_body(r_ref, d_ref, o_ref):
    o_ref[...] = r_ref[...] + d_ref[...]


def kernel(users, items, user_emb, item_emb, user_bias, item_bias):
    users = users.astype(jnp.int32)
    items = items.astype(jnp.int32)
    r, d = _sc_gather_dot(users, items, user_emb.T, item_emb.T,
                          user_bias.reshape(-1), item_bias.reshape(-1))
    out = pl.pallas_call(
        _tc_body,
        out_shape=jax.ShapeDtypeStruct((B, B), jnp.float32),
    )(r.reshape(B, 1), d.reshape(1, B))
    return out


# speedup vs baseline: 21.6229x; 2.6019x over previous
import functools

import jax
import jax.numpy as jnp
from jax import lax
from jax.experimental import pallas as pl
from jax.experimental.pallas import tpu as pltpu
from jax.experimental.pallas import tpu_sc as plsc

B = 1024
F = 64
WIN = 128
NBUF = 4
NC = 2
NS = 16
NW = NC * NS
BPW = B // NW

_mesh = plsc.VectorSubcoreMesh(core_axis_name="c", subcore_axis_name="s")

_GATHER_DN = lax.GatherDimensionNumbers(
    offset_dims=(), collapsed_slice_dims=(0,), start_index_map=(0,))


def _permute(x, idx):
    return lax.gather(x, idx[:, None], _GATHER_DN, (1,),
                      mode=lax.GatherScatterMode.PROMISE_IN_BOUNDS)


@functools.partial(
    pl.kernel,
    mesh=_mesh,
    out_type=[
        jax.ShapeDtypeStruct((B,), jnp.float32),
        jax.ShapeDtypeStruct((B,), jnp.float32),
    ],
    scratch_types=[
        pltpu.VMEM((BPW,), jnp.int32),
        pltpu.VMEM((BPW,), jnp.int32),
        pltpu.VMEM((NBUF, F, WIN), jnp.float32),
        pltpu.VMEM((NBUF, F, WIN), jnp.float32),
        pltpu.VMEM((NBUF, 1, WIN), jnp.float32),
        pltpu.VMEM((NBUF, 1, WIN), jnp.float32),
        pltpu.VMEM((BPW,), jnp.float32),
        pltpu.VMEM((BPW,), jnp.float32),
        pltpu.SemaphoreType.DMA,
    ],
    compiler_params=pltpu.CompilerParams(needs_layout_passes=False),
)
def _sc_gather_dot(users_hbm, items_hbm, uembt_hbm, iembt_hbm, ub_hbm, ib_hbm,
                   r_hbm, d_hbm,
                   uidx, iidx, ublk, iblk, ubb, ibb, rloc, dloc, sem):
    wid = lax.axis_index("s") * NC + lax.axis_index("c")
    base = wid * BPW
    pltpu.sync_copy(users_hbm.at[pl.ds(base, BPW)], uidx)
    pltpu.sync_copy(items_hbm.at[pl.ds(base, BPW)], iidx)
    iota = lax.iota(jnp.int32, 16)
    zeros = jnp.zeros((16,), jnp.int32)
    uvecs = [uidx[pl.ds(0, 16)], uidx[pl.ds(16, 16)]]
    ivecs = [iidx[pl.ds(0, 16)], iidx[pl.ds(16, 16)]]

    def issue(j):
        ru = uvecs[j // 16][j % 16]
        ri = ivecs[j // 16][j % 16]
        off_u = pl.multiple_of(ru & -WIN, WIN)
        off_i = pl.multiple_of(ri & -WIN, WIN)
        s = j % NBUF
        return (pltpu.async_copy(uembt_hbm.at[:, pl.ds(off_u, WIN)],
                                 ublk.at[s], sem),
                pltpu.async_copy(iembt_hbm.at[:, pl.ds(off_i, WIN)],
                                 iblk.at[s], sem),
                pltpu.async_copy(ub_hbm.at[:, pl.ds(off_u, WIN)],
                                 ubb.at[s], sem),
                pltpu.async_copy(ib_hbm.at[:, pl.ds(off_i, WIN)],
                                 ibb.at[s], sem))

    pending = [issue(j) for j in range(NBUF - 1)]
    dvecs = [jnp.zeros((16,), jnp.float32), jnp.zeros((16,), jnp.float32)]
    rvecs = [jnp.zeros((16,), jnp.float32), jnp.zeros((16,), jnp.float32)]
    for j in range(BPW):
        if j + NBUF - 1 < BPW:
            pending.append(issue(j + NBUF - 1))
        for cp in pending[j]:
            cp.wait()
        s = j % NBUF
        cu = zeros + (uvecs[j // 16][j % 16] & (WIN - 1))
        ci = zeros + (ivecs[j // 16][j % 16] & (WIN - 1))
        acc = jnp.zeros((16,), jnp.float32)
        for k in range(F // 16):
            rows = iota + k * 16
            acc = acc + (plsc.load_gather(ublk.at[s], [rows, cu])
                         * plsc.load_gather(iblk.at[s], [rows, ci]))
        for sh in (8, 4, 2, 1):
            acc = acc + _permute(acc, iota ^ sh)
        rcon = (plsc.load_gather(ubb.at[s], [zeros, cu])
                + plsc.load_gather(ibb.at[s], [zeros, ci]))
        lane = iota == (j % 16)
        dvecs[j // 16] = jnp.where(lane, acc, dvecs[j // 16])
        rvecs[j // 16] = jnp.where(lane, rcon, rvecs[j // 16])
    dloc[pl.ds(0, 16)] = dvecs[0]
    dloc[pl.ds(16, 16)] = dvecs[1]
    rloc[pl.ds(0, 16)] = rvecs[0]
    rloc[pl.ds(16, 16)] = rvecs[1]
    pltpu.sync_copy(rloc, r_hbm.at[pl.ds(base, BPW)])
    pltpu.sync_copy(dloc, d_hbm.at[pl.ds(base, BPW)])


def _tc_body(r_ref, d_ref, o_ref):
    o_ref[...] = r_ref[...] + d_ref[...]


def kernel(users, items, user_emb, item_emb, user_bias, item_bias):
    users = users.astype(jnp.int32)
    items = items.astype(jnp.int32)
    r, d = _sc_gather_dot(users, items, user_emb.T, item_emb.T,
                          user_bias.T, item_bias.T)
    out = pl.pallas_call(
        _tc_body,
        out_shape=jax.ShapeDtypeStruct((B, B), jnp.float32),
    )(r.reshape(B, 1), d.reshape(1, B))
    return out
